# baseline (speedup 1.0000x reference)
import functools

import jax
import jax.numpy as jnp
from jax import lax
from jax.experimental import pallas as pl
from jax.experimental.pallas import tpu as pltpu
from jax.experimental.pallas import tpu_sc as plsc

_EMBED = 768
_NEXP = 8
_BLK = 4096

_LANES = 16
_NW = 32
_CHUNK = 16
_SC_TOKENS = 4096


def _tc_router_body(x_ref, w_ref, g_ref, i_ref, tl_ref, load_ref):
    logits = jax.lax.dot_general(
        w_ref[:],
        x_ref[:],
        (((1,), (1,)), ((), ())),
        preferred_element_type=jnp.float32,
    )
    eidx = jax.lax.broadcasted_iota(jnp.int32, logits.shape, 0)
    neg = jnp.float32(-jnp.inf)

    l1 = jnp.max(logits, axis=0, keepdims=True)
    i1 = jnp.min(jnp.where(logits == l1, eidx, _NEXP), axis=0, keepdims=True)
    masked2 = jnp.where(eidx == i1, neg, logits)
    l2 = jnp.max(masked2, axis=0, keepdims=True)
    i2 = jnp.min(jnp.where(masked2 == l2, eidx, _NEXP), axis=0, keepdims=True)

    e21 = jnp.exp(l2 - l1)
    denom = 1.0 + e21
    g1 = 1.0 / denom
    g2 = e21 / denom

    g_ref[0:1, :] = g1
    g_ref[1:2, :] = g2
    i_ref[0:1, :] = i1
    i_ref[1:2, :] = i2
    tl_ref[0:1, :] = l1
    tl_ref[1:2, :] = l2

    part = jnp.sum(
        jnp.where(eidx == i1, g1, 0.0) + jnp.where(eidx == i2, g2, 0.0),
        axis=1,
        keepdims=True,
    )

    @pl.when(pl.program_id(0) == 0)
    def _init():
        load_ref[:] = jnp.zeros_like(load_ref)

    load_ref[:, 0:1] += part


def _tc_router(x2, W, n_tc):
    grid = (n_tc // _BLK,)
    return pl.pallas_call(
        _tc_router_body,
        grid=grid,
        in_specs=[
            pl.BlockSpec((_BLK, _EMBED), lambda i: (i, 0)),
            pl.BlockSpec((_NEXP, _EMBED), lambda i: (0, 0)),
        ],
        out_specs=[
            pl.BlockSpec((2, _BLK), lambda i: (0, i)),
            pl.BlockSpec((2, _BLK), lambda i: (0, i)),
            pl.BlockSpec((2, _BLK), lambda i: (0, i)),
            pl.BlockSpec((_NEXP, 128), lambda i: (0, 0)),
        ],
        out_shape=[
            jax.ShapeDtypeStruct((2, n_tc), jnp.float32),
            jax.ShapeDtypeStruct((2, n_tc), jnp.int32),
            jax.ShapeDtypeStruct((2, n_tc), jnp.float32),
            jax.ShapeDtypeStruct((_NEXP, 128), jnp.float32),
        ],
    )(x2[:n_tc], W)


def _make_sc_router(n_total, f_tokens):
    tpt = f_tokens // _NW
    base0 = n_total - f_tokens
    n_chunks = tpt // _CHUNK
    mesh = plsc.VectorSubcoreMesh(core_axis_name="c", subcore_axis_name="s")

    out_type = [
        jax.ShapeDtypeStruct((f_tokens,), jnp.float32),
        jax.ShapeDtypeStruct((f_tokens,), jnp.float32),
        jax.ShapeDtypeStruct((f_tokens,), jnp.int32),
        jax.ShapeDtypeStruct((f_tokens,), jnp.int32),
        jax.ShapeDtypeStruct((f_tokens,), jnp.float32),
        jax.ShapeDtypeStruct((f_tokens,), jnp.float32),
        jax.ShapeDtypeStruct((_NW, _NEXP, _LANES), jnp.float32),
    ]
    scratch = [
        pltpu.VMEM((_NEXP, _EMBED), jnp.float32),
        pltpu.VMEM((_CHUNK, _EMBED), jnp.float32),
        pltpu.VMEM((_CHUNK * _NEXP * _LANES,), jnp.float32),
        pltpu.VMEM((tpt,), jnp.float32),
        pltpu.VMEM((tpt,), jnp.float32),
        pltpu.VMEM((tpt,), jnp.int32),
        pltpu.VMEM((tpt,), jnp.int32),
        pltpu.VMEM((tpt,), jnp.float32),
        pltpu.VMEM((tpt,), jnp.float32),
        pltpu.VMEM((_NEXP, _LANES), jnp.float32),
    ]
    n_dchunks = _EMBED // _LANES
    tok_grp = 4

    def _bf16_round(v):
        u = plsc.bitcast(v, jnp.uint32)
        lsb = (u >> 16) & jnp.uint32(1)
        r = (u + jnp.uint32(0x7FFF) + lsb) & jnp.uint32(0xFFFF0000)
        return plsc.bitcast(r, jnp.float32)

    @functools.partial(
        pl.kernel, out_type=out_type, mesh=mesh, scratch_types=scratch,
        compiler_params=pltpu.CompilerParams(needs_layout_passes=False),
    )
    def sc_router(x_hbm, w_hbm, g1_o, g2_o, i1_o, i2_o, l1_o, l2_o, load_o,
                  w_v, x_v, red_b, g1_b, g2_b, i1_b, i2_b, l1_b, l2_b,
                  load_b):
        wid = lax.axis_index("s") * 2 + lax.axis_index("c")
        tok0 = base0 + wid * tpt
        pltpu.sync_copy(w_hbm, w_v)

        lanes_i = lax.iota(jnp.int32, _LANES)
        zero_f = jnp.zeros((_LANES,), jnp.float32)

        for e in range(_NEXP):

            def w_round(c, carry, e=e):
                sl = pl.ds(c * _LANES, _LANES)
                w_v[e, sl] = _bf16_round(w_v[e, sl])
                return carry

            lax.fori_loop(0, n_dchunks, w_round, 0)
        for e in range(_NEXP):
            load_b[e, :] = zero_f

        def chunk_body(c, carry):
            pltpu.sync_copy(x_hbm.at[pl.ds(tok0 + c * _CHUNK, _CHUNK)], x_v)

            for tg in range(_CHUNK // tok_grp):

                def d_body(dc, accs):
                    base = dc * _LANES
                    xs = tuple(
                        _bf16_round(x_v[tg * tok_grp + t, pl.ds(base, _LANES)])
                        for t in range(tok_grp)
                    )
                    ws = tuple(
                        w_v[e, pl.ds(base, _LANES)] for e in range(_NEXP)
                    )
                    return tuple(
                        accs[t * _NEXP + e] + xs[t] * ws[e]
                        for t in range(tok_grp)
                        for e in range(_NEXP)
                    )

                accs = lax.fori_loop(
                    0, n_dchunks, d_body,
                    tuple(zero_f for _ in range(tok_grp * _NEXP)),
                )
                for t in range(tok_grp):
                    for e in range(_NEXP):
                        row = e * _CHUNK + tg * tok_grp + t
                        red_b[pl.ds(row * _LANES, _LANES)] = (
                            accs[t * _NEXP + e]
                        )

            lanes16 = lanes_i * _LANES
            logits = []
            for e in range(_NEXP):
                tot = zero_f
                for j in range(_LANES):
                    idxs = lanes16 + (e * _CHUNK * _LANES + j)
                    tot = tot + plsc.load_gather(red_b, [idxs])
                logits.append(tot)

            m1 = logits[0]
            i1 = jnp.zeros((_LANES,), jnp.int32)
            m2 = jnp.full((_LANES,), -jnp.inf, jnp.float32)
            i2 = jnp.zeros((_LANES,), jnp.int32)
            for e in range(1, _NEXP):
                v = logits[e]
                ei = jnp.full((_LANES,), e, jnp.int32)
                gt1 = v > m1
                gt2 = v > m2
                m2 = jnp.where(gt1, m1, jnp.where(gt2, v, m2))
                i2 = jnp.where(gt1, i1, jnp.where(gt2, ei, i2))
                m1 = jnp.where(gt1, v, m1)
                i1 = jnp.where(gt1, ei, i1)

            e21 = jnp.exp(m2 - m1)
            denom = 1.0 + e21
            g1 = 1.0 / denom
            g2 = e21 / denom

            sl = pl.ds(c * _CHUNK, _CHUNK)
            g1_b[sl] = g1
            g2_b[sl] = g2
            i1_b[sl] = i1
            i2_b[sl] = i2
            l1_b[sl] = m1
            l2_b[sl] = m2
            for e in range(_NEXP):
                load_b[e, :] = (
                    load_b[e, :]
                    + jnp.where(i1 == e, g1, 0.0)
                    + jnp.where(i2 == e, g2, 0.0)
                )
            return carry

        lax.fori_loop(0, n_chunks, chunk_body, 0)

        out0 = wid * tpt
        pltpu.sync_copy(g1_b, g1_o.at[pl.ds(out0, tpt)])
        pltpu.sync_copy(g2_b, g2_o.at[pl.ds(out0, tpt)])
        pltpu.sync_copy(i1_b, i1_o.at[pl.ds(out0, tpt)])
        pltpu.sync_copy(i2_b, i2_o.at[pl.ds(out0, tpt)])
        pltpu.sync_copy(l1_b, l1_o.at[pl.ds(out0, tpt)])
        pltpu.sync_copy(l2_b, l2_o.at[pl.ds(out0, tpt)])
        pltpu.sync_copy(load_b, load_o.at[wid])

    return sc_router


@jax.jit
def kernel(x, W):
    b, s, d = x.shape
    n = b * s
    x2 = x.reshape(n, d)
    n_tc = n - _SC_TOKENS

    g_t, i_t, tl_t, load_tc = _tc_router(x2, W, n_tc)
    sg1, sg2, si1, si2, sl1, sl2, load_sc = _make_sc_router(n, _SC_TOKENS)(
        x2, W
    )

    gates = jnp.concatenate(
        [g_t.T, jnp.stack([sg1, sg2], axis=-1)], axis=0
    ).reshape(b, s, 2)
    idx = jnp.concatenate(
        [i_t.T, jnp.stack([si1, si2], axis=-1)], axis=0
    ).reshape(b, s, 2)
    tl = jnp.concatenate(
        [tl_t.T, jnp.stack([sl1, sl2], axis=-1)], axis=0
    ).reshape(b, s, 2)
    load = load_tc[:, 0] + jnp.sum(load_sc, axis=(0, 2))
    return gates, idx, load, tl


# speedup vs baseline: 1.0001x; 1.0001x over previous
import functools

import jax
import jax.numpy as jnp
from jax import lax
from jax.experimental import pallas as pl
from jax.experimental.pallas import tpu as pltpu
from jax.experimental.pallas import tpu_sc as plsc

_EMBED = 768
_NEXP = 8
_BLK = 4096

_LANES = 16
_NW = 32
_CHUNK = 16
_SC_TOKENS = 4096


def _tc_router_body(x_ref, w_ref, g_ref, i_ref, tl_ref, load_ref):
    logits = jax.lax.dot_general(
        w_ref[:],
        x_ref[:],
        (((1,), (1,)), ((), ())),
        preferred_element_type=jnp.float32,
    )
    eidx = jax.lax.broadcasted_iota(jnp.int32, logits.shape, 0)
    neg = jnp.float32(-jnp.inf)

    l1 = jnp.max(logits, axis=0, keepdims=True)
    i1 = jnp.min(jnp.where(logits == l1, eidx, _NEXP), axis=0, keepdims=True)
    masked2 = jnp.where(eidx == i1, neg, logits)
    l2 = jnp.max(masked2, axis=0, keepdims=True)
    i2 = jnp.min(jnp.where(masked2 == l2, eidx, _NEXP), axis=0, keepdims=True)

    e21 = jnp.exp(l2 - l1)
    denom = 1.0 + e21
    g1 = 1.0 / denom
    g2 = e21 / denom

    g_ref[0:1, :] = g1
    g_ref[1:2, :] = g2
    i_ref[0:1, :] = i1
    i_ref[1:2, :] = i2
    tl_ref[0:1, :] = l1
    tl_ref[1:2, :] = l2

    part = jnp.sum(
        jnp.where(eidx == i1, g1, 0.0) + jnp.where(eidx == i2, g2, 0.0),
        axis=1,
        keepdims=True,
    )

    @pl.when(pl.program_id(0) == 0)
    def _init():
        load_ref[:] = jnp.zeros_like(load_ref)

    load_ref[:, 0:1] += part


def _tc_router(x2, W, n_tc):
    grid = (n_tc // _BLK,)
    return pl.pallas_call(
        _tc_router_body,
        grid=grid,
        in_specs=[
            pl.BlockSpec((_BLK, _EMBED), lambda i: (i, 0)),
            pl.BlockSpec((_NEXP, _EMBED), lambda i: (0, 0)),
        ],
        out_specs=[
            pl.BlockSpec((2, _BLK), lambda i: (0, i)),
            pl.BlockSpec((2, _BLK), lambda i: (0, i)),
            pl.BlockSpec((2, _BLK), lambda i: (0, i)),
            pl.BlockSpec((_NEXP, 128), lambda i: (0, 0)),
        ],
        out_shape=[
            jax.ShapeDtypeStruct((2, n_tc), jnp.float32),
            jax.ShapeDtypeStruct((2, n_tc), jnp.int32),
            jax.ShapeDtypeStruct((2, n_tc), jnp.float32),
            jax.ShapeDtypeStruct((_NEXP, 128), jnp.float32),
        ],
    )(x2[:n_tc], W)


def _make_sc_router(n_total, f_tokens):
    tpt = f_tokens // _NW
    base0 = n_total - f_tokens
    n_chunks = tpt // _CHUNK
    mesh = plsc.VectorSubcoreMesh(core_axis_name="c", subcore_axis_name="s")

    out_type = [
        jax.ShapeDtypeStruct((f_tokens,), jnp.float32),
        jax.ShapeDtypeStruct((f_tokens,), jnp.float32),
        jax.ShapeDtypeStruct((f_tokens,), jnp.int32),
        jax.ShapeDtypeStruct((f_tokens,), jnp.int32),
        jax.ShapeDtypeStruct((f_tokens,), jnp.float32),
        jax.ShapeDtypeStruct((f_tokens,), jnp.float32),
        jax.ShapeDtypeStruct((_NW, _NEXP, _LANES), jnp.float32),
    ]
    scratch = [
        pltpu.VMEM((_NEXP, _EMBED), jnp.float32),
        pltpu.VMEM((_CHUNK, _EMBED), jnp.float32),
        pltpu.VMEM((_CHUNK * _NEXP * _LANES,), jnp.float32),
        pltpu.VMEM((tpt,), jnp.float32),
        pltpu.VMEM((tpt,), jnp.float32),
        pltpu.VMEM((tpt,), jnp.int32),
        pltpu.VMEM((tpt,), jnp.int32),
        pltpu.VMEM((tpt,), jnp.float32),
        pltpu.VMEM((tpt,), jnp.float32),
        pltpu.VMEM((_NEXP, _LANES), jnp.float32),
    ]
    n_dchunks = _EMBED // _LANES
    tok_grp = 4

    def _bf16_round(v):
        u = plsc.bitcast(v, jnp.uint32)
        lsb = (u >> 16) & jnp.uint32(1)
        r = (u + jnp.uint32(0x7FFF) + lsb) & jnp.uint32(0xFFFF0000)
        return plsc.bitcast(r, jnp.float32)

    @functools.partial(
        pl.kernel, out_type=out_type, mesh=mesh, scratch_types=scratch,
        compiler_params=pltpu.CompilerParams(needs_layout_passes=False),
    )
    def sc_router(x_hbm, w_hbm, g1_o, g2_o, i1_o, i2_o, l1_o, l2_o, load_o,
                  w_v, x_v, red_b, g1_b, g2_b, i1_b, i2_b, l1_b, l2_b,
                  load_b):
        wid = lax.axis_index("s") * 2 + lax.axis_index("c")
        tok0 = base0 + wid * tpt
        pltpu.sync_copy(w_hbm, w_v)

        lanes_i = lax.iota(jnp.int32, _LANES)
        zero_f = jnp.zeros((_LANES,), jnp.float32)

        for e in range(_NEXP):

            def w_round(c, carry, e=e):
                sl = pl.ds(c * _LANES, _LANES)
                w_v[e, sl] = _bf16_round(w_v[e, sl])
                return carry

            lax.fori_loop(0, n_dchunks, w_round, 0)
        for e in range(_NEXP):
            load_b[e, :] = zero_f

        def chunk_body(c, carry):
            pltpu.sync_copy(x_hbm.at[pl.ds(tok0 + c * _CHUNK, _CHUNK)], x_v)

            for tg in range(_CHUNK // tok_grp):

                def d_body(dc, accs):
                    base = dc * _LANES
                    xs = tuple(
                        _bf16_round(x_v[tg * tok_grp + t, pl.ds(base, _LANES)])
                        for t in range(tok_grp)
                    )
                    ws = tuple(
                        w_v[e, pl.ds(base, _LANES)] for e in range(_NEXP)
                    )
                    return tuple(
                        accs[t * _NEXP + e] + xs[t] * ws[e]
                        for t in range(tok_grp)
                        for e in range(_NEXP)
                    )

                accs = lax.fori_loop(
                    0, n_dchunks, d_body,
                    tuple(zero_f for _ in range(tok_grp * _NEXP)),
                )
                for t in range(tok_grp):
                    for e in range(_NEXP):
                        row = e * _CHUNK + tg * tok_grp + t
                        red_b[pl.ds(row * _LANES, _LANES)] = (
                            accs[t * _NEXP + e]
                        )

            lanes16 = lanes_i * _LANES
            logits = []
            for e in range(_NEXP):
                tot = zero_f
                for j in range(_LANES):
                    idxs = lanes16 + (e * _CHUNK * _LANES + j)
                    tot = tot + plsc.load_gather(red_b, [idxs])
                logits.append(tot)

            m1 = logits[0]
            i1 = jnp.zeros((_LANES,), jnp.int32)
            m2 = jnp.full((_LANES,), -jnp.inf, jnp.float32)
            i2 = jnp.zeros((_LANES,), jnp.int32)
            for e in range(1, _NEXP):
                v = logits[e]
                ei = jnp.full((_LANES,), e, jnp.int32)
                gt1 = v > m1
                gt2 = v > m2
                m2 = jnp.where(gt1, m1, jnp.where(gt2, v, m2))
                i2 = jnp.where(gt1, i1, jnp.where(gt2, ei, i2))
                m1 = jnp.where(gt1, v, m1)
                i1 = jnp.where(gt1, ei, i1)

            e21 = jnp.exp(m2 - m1)
            denom = 1.0 + e21
            g1 = 1.0 / denom
            g2 = e21 / denom

            sl = pl.ds(c * _CHUNK, _CHUNK)
            g1_b[sl] = g1
            g2_b[sl] = g2
            i1_b[sl] = i1
            i2_b[sl] = i2
            l1_b[sl] = m1
            l2_b[sl] = m2
            for e in range(_NEXP):
                load_b[e, :] = (
                    load_b[e, :]
                    + jnp.where(i1 == e, g1, 0.0)
                    + jnp.where(i2 == e, g2, 0.0)
                )
            return carry

        lax.fori_loop(0, n_chunks, chunk_body, 0)

        out0 = wid * tpt
        pltpu.sync_copy(g1_b, g1_o.at[pl.ds(out0, tpt)])
        pltpu.sync_copy(g2_b, g2_o.at[pl.ds(out0, tpt)])
        pltpu.sync_copy(i1_b, i1_o.at[pl.ds(out0, tpt)])
        pltpu.sync_copy(i2_b, i2_o.at[pl.ds(out0, tpt)])
        pltpu.sync_copy(l1_b, l1_o.at[pl.ds(out0, tpt)])
        pltpu.sync_copy(l2_b, l2_o.at[pl.ds(out0, tpt)])
        pltpu.sync_copy(load_b, load_o.at[wid])

    return sc_router


@jax.jit
def kernel(x, W):
    b, s, d = x.shape
    n = b * s
    x2 = x.reshape(n, d)
    n_tc = n - _SC_TOKENS

    sg1, sg2, si1, si2, sl1, sl2, load_sc = _make_sc_router(n, _SC_TOKENS)(
        x2, W
    )
    g_t, i_t, tl_t, load_tc = _tc_router(x2, W, n_tc)

    gates = jnp.concatenate(
        [g_t.T, jnp.stack([sg1, sg2], axis=-1)], axis=0
    ).reshape(b, s, 2)
    idx = jnp.concatenate(
        [i_t.T, jnp.stack([si1, si2], axis=-1)], axis=0
    ).reshape(b, s, 2)
    tl = jnp.concatenate(
        [tl_t.T, jnp.stack([sl1, sl2], axis=-1)], axis=0
    ).reshape(b, s, 2)
    load = load_tc[:, 0] + jnp.sum(load_sc, axis=(0, 2))
    return gates, idx, load, tl


# speedup vs baseline: 2.9423x; 2.9422x over previous
import jax
import jax.numpy as jnp
from jax.experimental import pallas as pl
from jax.experimental.pallas import tpu as pltpu

_EMBED = 768
_NEXP = 8
_BLK = 4096
_DSPLIT = 2
_DHALF = _EMBED // _DSPLIT


def _router_body(x_ref, w_ref, g_ref, i_ref, tl_ref, load_ref, acc_ref):
    j = pl.program_id(1)
    part = jax.lax.dot_general(
        w_ref[:],
        x_ref[:],
        (((1,), (1,)), ((), ())),
        preferred_element_type=jnp.float32,
    )

    @pl.when(j == 0)
    def _first():
        acc_ref[:] = part

    @pl.when(j == _DSPLIT - 1)
    def _last():
        logits = acc_ref[:] + part
        eidx = jax.lax.broadcasted_iota(jnp.int32, logits.shape, 0)
        neg = jnp.float32(-jnp.inf)

        l1 = jnp.max(logits, axis=0, keepdims=True)
        i1 = jnp.min(
            jnp.where(logits == l1, eidx, _NEXP), axis=0, keepdims=True
        )
        masked2 = jnp.where(eidx == i1, neg, logits)
        l2 = jnp.max(masked2, axis=0, keepdims=True)
        i2 = jnp.min(
            jnp.where(masked2 == l2, eidx, _NEXP), axis=0, keepdims=True
        )

        e21 = jnp.exp(l2 - l1)
        denom = 1.0 + e21
        g1 = 1.0 / denom
        g2 = e21 / denom

        g_ref[0:1, :] = g1
        g_ref[1:2, :] = g2
        i_ref[0:1, :] = i1
        i_ref[1:2, :] = i2
        tl_ref[0:1, :] = l1
        tl_ref[1:2, :] = l2

        part_load = jnp.sum(
            jnp.where(eidx == i1, g1, 0.0) + jnp.where(eidx == i2, g2, 0.0),
            axis=1,
            keepdims=True,
        )

        @pl.when(pl.program_id(0) == 0)
        def _init():
            load_ref[:] = jnp.zeros_like(load_ref)

        load_ref[:, 0:1] += part_load


@jax.jit
def kernel(x, W):
    b, s, d = x.shape
    n = b * s
    x2 = x.reshape(n, d)

    grid = (n // _BLK, _DSPLIT)
    g_t, i_t, tl_t, load = pl.pallas_call(
        _router_body,
        grid=grid,
        in_specs=[
            pl.BlockSpec((_BLK, _DHALF), lambda i, j: (i, j)),
            pl.BlockSpec((_NEXP, _DHALF), lambda i, j: (0, j)),
        ],
        out_specs=[
            pl.BlockSpec((2, _BLK), lambda i, j: (0, i)),
            pl.BlockSpec((2, _BLK), lambda i, j: (0, i)),
            pl.BlockSpec((2, _BLK), lambda i, j: (0, i)),
            pl.BlockSpec((_NEXP, 128), lambda i, j: (0, 0)),
        ],
        out_shape=[
            jax.ShapeDtypeStruct((2, n), jnp.float32),
            jax.ShapeDtypeStruct((2, n), jnp.int32),
            jax.ShapeDtypeStruct((2, n), jnp.float32),
            jax.ShapeDtypeStruct((_NEXP, 128), jnp.float32),
        ],
        scratch_shapes=[pltpu.VMEM((_NEXP, _BLK), jnp.float32)],
    )(x2, W)

    return (
        g_t.T.reshape(b, s, 2),
        i_t.T.reshape(b, s, 2),
        load[:, 0],
        tl_t.T.reshape(b, s, 2),
    )


# speedup vs baseline: 3.1973x; 1.0867x over previous
import jax
import jax.numpy as jnp
from jax.experimental import pallas as pl

_EMBED = 768
_NEXP = 8
_BLK = 4096


def _router_body(x_ref, w_ref, g_ref, i_ref, tl_ref, load_ref):
    logits = jax.lax.dot_general(
        w_ref[:],
        x_ref[:],
        (((1,), (1,)), ((), ())),
        preferred_element_type=jnp.float32,
    )
    eidx = jax.lax.broadcasted_iota(jnp.int32, logits.shape, 0)
    neg = jnp.float32(-jnp.inf)

    l1 = jnp.max(logits, axis=0, keepdims=True)
    i1 = jnp.min(jnp.where(logits == l1, eidx, _NEXP), axis=0, keepdims=True)
    masked2 = jnp.where(eidx == i1, neg, logits)
    l2 = jnp.max(masked2, axis=0, keepdims=True)
    i2 = jnp.min(jnp.where(masked2 == l2, eidx, _NEXP), axis=0, keepdims=True)

    e21 = jnp.exp(l2 - l1)
    denom = 1.0 + e21
    g1 = 1.0 / denom
    g2 = e21 / denom

    g_ref[0:1, :] = g1
    g_ref[1:2, :] = g2
    i_ref[0:1, :] = i1
    i_ref[1:2, :] = i2
    tl_ref[0:1, :] = l1
    tl_ref[1:2, :] = l2

    part = jnp.sum(
        jnp.where(eidx == i1, g1, 0.0) + jnp.where(eidx == i2, g2, 0.0),
        axis=1,
        keepdims=True,
    )

    @pl.when(pl.program_id(0) == 0)
    def _init():
        load_ref[:] = jnp.zeros_like(load_ref)

    load_ref[:, 0:1] += part


@jax.jit
def kernel(x, W):
    b, s, d = x.shape
    n = b * s
    x2 = x.reshape(n, d)

    grid = (n // _BLK,)
    g_t, i_t, tl_t, load = pl.pallas_call(
        _router_body,
        grid=grid,
        in_specs=[
            pl.BlockSpec((_BLK, d), lambda i: (i, 0)),
            pl.BlockSpec((_NEXP, d), lambda i: (0, 0)),
        ],
        out_specs=[
            pl.BlockSpec((2, _BLK), lambda i: (0, i)),
            pl.BlockSpec((2, _BLK), lambda i: (0, i)),
            pl.BlockSpec((2, _BLK), lambda i: (0, i)),
            pl.BlockSpec((_NEXP, 128), lambda i: (0, 0)),
        ],
        out_shape=[
            jax.ShapeDtypeStruct((2, n), jnp.float32),
            jax.ShapeDtypeStruct((2, n), jnp.int32),
            jax.ShapeDtypeStruct((2, n), jnp.float32),
            jax.ShapeDtypeStruct((_NEXP, 128), jnp.float32),
        ],
    )(x2, W)

    return (
        g_t.T.reshape(b, s, 2),
        i_t.T.reshape(b, s, 2),
        load[:, 0],
        tl_t.T.reshape(b, s, 2),
    )
